# baseline (speedup 1.0000x reference)
import functools

import jax
import jax.numpy as jnp
import numpy as np
from jax import lax
from jax.experimental import pallas as pl
from jax.experimental.pallas import tpu as pltpu
from jax.experimental.pallas import tpu_sc as plsc

H = 1024
E = 8
K = 2
T = 4096
INTER = int(H * 8 // 3)
GH = H // 2
CAP = int(1.25 * T * K / E)

BT = 512
TB = T // BT
BC = 640
CB = CAP // BC


def _gate_kernel(x_ref, wg1x_ref, wg1s_ref, bg1_ref, wg2_ref, w_ref):
    x = x_ref[...]
    mean = jnp.mean(x, axis=-1, keepdims=True)
    var = jnp.mean((x - mean) ** 2, axis=-1, keepdims=True)
    std = jnp.sqrt(var)
    mx = jnp.max(x, axis=-1, keepdims=True)
    mn = jnp.min(x, axis=-1, keepdims=True)
    am = jnp.mean(jnp.abs(x), axis=-1, keepdims=True)
    nrm = jnp.sqrt(jnp.sum(x * x, axis=-1, keepdims=True)) * (1.0 / np.sqrt(H))
    stats = jnp.concatenate([mean, std, mx, mn, am, nrm], axis=-1)

    g = (
        jnp.dot(x, wg1x_ref[...], preferred_element_type=jnp.float32)
        + jnp.dot(stats, wg1s_ref[...], preferred_element_type=jnp.float32)
        + bg1_ref[...]
    )
    g = jax.nn.gelu(g)
    logits = jnp.dot(g, wg2_ref[...], preferred_element_type=jnp.float32)

    m = jnp.max(logits, axis=-1, keepdims=True)
    ex = jnp.exp(logits - m)
    p = ex / jnp.sum(ex, axis=-1, keepdims=True)

    col = jax.lax.broadcasted_iota(jnp.int32, p.shape, 1)
    m1 = jnp.max(p, axis=1, keepdims=True)
    i1 = jnp.min(jnp.where(p == m1, col, E), axis=1, keepdims=True)
    is1 = col == i1
    p2m = jnp.where(is1, -jnp.inf, p)
    m2 = jnp.max(p2m, axis=1, keepdims=True)
    i2 = jnp.min(jnp.where(p2m == m2, col, E), axis=1, keepdims=True)
    w_ref[...] = jnp.where(is1 | (col == i2), p, 0.0)


def _route_kernel(w_ref, rk_ref, cidx_ref, wc_ref):
    w = w_ref[...]
    rcols = []
    for e in range(E):
        colv = w[:, e : e + 1]
        row = jnp.transpose(colv)
        sidx = jax.lax.broadcasted_iota(jnp.int32, (1, T), 1)
        wchunks = []
        rchunks = []
        for c in range(T // BT):
            wt = colv[c * BT : (c + 1) * BT]
            tidx = jax.lax.broadcasted_iota(jnp.int32, (BT, 1), 0) + c * BT
            beats = (row > wt) | ((row == wt) & (sidx < tidx))
            rank = jnp.sum(beats.astype(jnp.int32), axis=1, keepdims=True)
            keep = (wt > 0) & (rank < CAP)
            wchunks.append(jnp.where(keep, wt, 0.0))
            rchunks.append(jnp.where(keep, rank, T))
        rcol = jnp.concatenate(rchunks, axis=0)
        rcols.append(rcol)
        rk_row = jnp.transpose(rcol)
        wm_row = jnp.transpose(jnp.concatenate(wchunks, axis=0))
        for c in range(CB):
            riota = jax.lax.broadcasted_iota(jnp.int32, (BC, 1), 0) + c * BC
            ohb = rk_row == riota
            cidx_ref[e, c * BC : (c + 1) * BC] = jnp.sum(
                jnp.where(ohb, sidx, 0), axis=1, keepdims=True
            )
            wc_ref[e, c * BC : (c + 1) * BC] = jnp.sum(
                jnp.where(ohb, wm_row, 0.0), axis=1, keepdims=True
            )
    rk_ref[...] = jnp.concatenate(rcols, axis=1)


GCH = 40


def _sc_gather(x_hbm, idx_hbm, out_hbm, idx_v0, idx_v1, rows_v0, rows_v1, sem0, sem1):
    info = plsc.get_sparse_core_info()
    nw = info.num_cores * info.num_subcores
    rpw = (E * CAP) // nw
    wid = lax.axis_index("s") * info.num_cores + lax.axis_index("c")
    base = wid * rpw
    idx_v = [idx_v0, idx_v1]
    rows_v = [rows_v0, rows_v1]
    sems = [sem0, sem1]
    n = rpw // GCH
    pending = [None] * 2
    for i in range(n):
        b = i % 2
        if pending[b] is not None:
            h, poff = pending[b]
            h.wait()
            pltpu.sync_copy(rows_v[b], out_hbm.at[pl.ds(poff, GCH)])
        off = base + i * GCH
        pltpu.sync_copy(idx_hbm.at[pl.ds(off, GCH)], idx_v[b])
        pending[b] = (pltpu.async_copy(x_hbm.at[idx_v[b]], rows_v[b], sems[b]), off)
    for b in range(2):
        if pending[b] is not None:
            h, poff = pending[b]
            h.wait()
            pltpu.sync_copy(rows_v[b], out_hbm.at[pl.ds(poff, GCH)])


def _ffn_kernel(xg_ref, w1a_ref, w1b_ref, w2_ref, wc_ref, ct_ref):
    xg = xg_ref[0].astype(jnp.bfloat16)
    h1a = jnp.dot(xg, w1a_ref[0], preferred_element_type=jnp.float32)
    h1b = jnp.dot(xg, w1b_ref[0], preferred_element_type=jnp.float32)
    act = (h1a * jax.nn.sigmoid(h1a) * h1b).astype(jnp.bfloat16)
    oe = jnp.dot(act, w2_ref[0], preferred_element_type=jnp.float32)
    ct_ref[0] = (oe * wc_ref[0]).astype(jnp.bfloat16)


def _scatter_kernel(rk_ref, ct_ref, out_ref):
    e = pl.program_id(0)
    rk = rk_ref[...]
    col = jax.lax.broadcasted_iota(jnp.int32, rk.shape, 1)
    rk_col = jnp.sum(jnp.where(col == e, rk, 0), axis=1, keepdims=True)
    riota = jax.lax.broadcasted_iota(jnp.int32, (1, CAP), 1)
    oht = (rk_col == riota).astype(jnp.bfloat16)
    delta = jnp.dot(oht, ct_ref[0], preferred_element_type=jnp.float32)

    @pl.when(e == 0)
    def _():
        out_ref[...] = delta

    @pl.when(e > 0)
    def _():
        out_ref[...] += delta


def kernel(x, Wg1, bg1, Wg2, W1, W2):
    w = pl.pallas_call(
        _gate_kernel,
        grid=(TB,),
        in_specs=[
            pl.BlockSpec((BT, H), lambda i: (i, 0)),
            pl.BlockSpec((H, GH), lambda i: (0, 0)),
            pl.BlockSpec((6, GH), lambda i: (0, 0)),
            pl.BlockSpec((1, GH), lambda i: (0, 0)),
            pl.BlockSpec((GH, E), lambda i: (0, 0)),
        ],
        out_specs=pl.BlockSpec((BT, E), lambda i: (i, 0)),
        out_shape=jax.ShapeDtypeStruct((T, E), jnp.float32),
    )(x, Wg1[:H], Wg1[H:], bg1[None, :], Wg2)

    rk, cidx, wc = pl.pallas_call(
        _route_kernel,
        out_shape=[
            jax.ShapeDtypeStruct((T, E), jnp.int32),
            jax.ShapeDtypeStruct((E, CAP, 1), jnp.int32),
            jax.ShapeDtypeStruct((E, CAP, 1), jnp.float32),
        ],
    )(w)

    mesh = plsc.VectorSubcoreMesh(core_axis_name="c", subcore_axis_name="s")
    info = plsc.get_sparse_core_info()
    rpw = (E * CAP) // (info.num_cores * info.num_subcores)
    xg_flat = functools.partial(
        pl.kernel,
        mesh=mesh,
        out_type=jax.ShapeDtypeStruct((E * CAP, H), jnp.float32),
        scratch_types=[
            pltpu.VMEM((GCH,), jnp.int32),
            pltpu.VMEM((GCH,), jnp.int32),
            pltpu.VMEM((GCH, H), jnp.float32),
            pltpu.VMEM((GCH, H), jnp.float32),
            pltpu.SemaphoreType.DMA,
            pltpu.SemaphoreType.DMA,
        ],
    )(_sc_gather)(x, cidx.reshape(E * CAP))
    xg = xg_flat.reshape(E, CAP, H)

    w1a = W1[:, :, :INTER].astype(jnp.bfloat16)
    w1b = W1[:, :, INTER:].astype(jnp.bfloat16)
    w2b = W2.astype(jnp.bfloat16)

    ct = pl.pallas_call(
        _ffn_kernel,
        grid=(E, CB),
        in_specs=[
            pl.BlockSpec((1, BC, H), lambda e, c: (e, c, 0)),
            pl.BlockSpec((1, H, INTER), lambda e, c: (e, 0, 0)),
            pl.BlockSpec((1, H, INTER), lambda e, c: (e, 0, 0)),
            pl.BlockSpec((1, INTER, H), lambda e, c: (e, 0, 0)),
            pl.BlockSpec((1, BC, 1), lambda e, c: (e, c, 0)),
        ],
        out_specs=pl.BlockSpec((1, BC, H), lambda e, c: (e, c, 0)),
        out_shape=jax.ShapeDtypeStruct((E, CAP, H), jnp.bfloat16),
    )(xg, w1a, w1b, w2b, wc)

    out = pl.pallas_call(
        _scatter_kernel,
        grid=(E,),
        in_specs=[
            pl.BlockSpec((T, E), lambda e: (0, 0)),
            pl.BlockSpec((1, CAP, H), lambda e: (e, 0, 0)),
        ],
        out_specs=pl.BlockSpec((T, H), lambda e: (0, 0)),
        out_shape=jax.ShapeDtypeStruct((T, H), jnp.float32),
    )(rk, ct)

    return out


# speedup vs baseline: 1.1342x; 1.1342x over previous
import jax
import jax.numpy as jnp
import numpy as np
from jax.experimental import pallas as pl

H = 1024
E = 8
K = 2
T = 4096
INTER = int(H * 8 // 3)
GH = H // 2
CAP = int(1.25 * T * K / E)

BT = 512
TB = T // BT
BC = 640
CB = CAP // BC


def _gate_kernel(x_ref, wg1x_ref, wg1s_ref, bg1_ref, wg2_ref, w_ref):
    x = x_ref[...]
    mean = jnp.mean(x, axis=-1, keepdims=True)
    var = jnp.mean((x - mean) ** 2, axis=-1, keepdims=True)
    std = jnp.sqrt(var)
    mx = jnp.max(x, axis=-1, keepdims=True)
    mn = jnp.min(x, axis=-1, keepdims=True)
    am = jnp.mean(jnp.abs(x), axis=-1, keepdims=True)
    nrm = jnp.sqrt(jnp.sum(x * x, axis=-1, keepdims=True)) * (1.0 / np.sqrt(H))
    stats = jnp.concatenate([mean, std, mx, mn, am, nrm], axis=-1)

    g = (
        jnp.dot(x, wg1x_ref[...], preferred_element_type=jnp.float32)
        + jnp.dot(stats, wg1s_ref[...], preferred_element_type=jnp.float32)
        + bg1_ref[...]
    )
    g = jax.nn.gelu(g)
    logits = jnp.dot(g, wg2_ref[...], preferred_element_type=jnp.float32)

    m = jnp.max(logits, axis=-1, keepdims=True)
    ex = jnp.exp(logits - m)
    p = ex / jnp.sum(ex, axis=-1, keepdims=True)

    col = jax.lax.broadcasted_iota(jnp.int32, p.shape, 1)
    m1 = jnp.max(p, axis=1, keepdims=True)
    i1 = jnp.min(jnp.where(p == m1, col, E), axis=1, keepdims=True)
    is1 = col == i1
    p2m = jnp.where(is1, -jnp.inf, p)
    m2 = jnp.max(p2m, axis=1, keepdims=True)
    i2 = jnp.min(jnp.where(p2m == m2, col, E), axis=1, keepdims=True)
    w_ref[...] = jnp.where(is1 | (col == i2), p, 0.0)


def _route_kernel(w_ref, wm_ref, rk_ref):
    w = w_ref[...]
    wcols = []
    rcols = []
    for e in range(E):
        colv = w[:, e : e + 1]
        row = jnp.transpose(colv)
        sidx = jax.lax.broadcasted_iota(jnp.int32, (1, T), 1)
        wchunks = []
        rchunks = []
        for c in range(T // BT):
            wt = colv[c * BT : (c + 1) * BT]
            tidx = jax.lax.broadcasted_iota(jnp.int32, (BT, 1), 0) + c * BT
            beats = (row > wt) | ((row == wt) & (sidx < tidx))
            rank = jnp.sum(beats.astype(jnp.int32), axis=1, keepdims=True)
            keep = (wt > 0) & (rank < CAP)
            wchunks.append(jnp.where(keep, wt, 0.0))
            rchunks.append(jnp.where(keep, rank, T))
        wcols.append(jnp.concatenate(wchunks, axis=0))
        rcols.append(jnp.concatenate(rchunks, axis=0))
    wm_ref[...] = jnp.concatenate(wcols, axis=1)
    rk_ref[...] = jnp.concatenate(rcols, axis=1)


def _gather_kernel(xbf_ref, rk_ref, wm_ref, xg_ref, wc_ref):
    e = pl.program_id(0)
    c = pl.program_id(1)
    rk = rk_ref[...]
    wm = wm_ref[...]
    col = jax.lax.broadcasted_iota(jnp.int32, rk.shape, 1)
    rk_col = jnp.sum(jnp.where(col == e, rk, 0), axis=1, keepdims=True)
    wm_col = jnp.sum(jnp.where(col == e, wm, 0.0), axis=1, keepdims=True)
    rk_row = jnp.transpose(rk_col)
    wm_row = jnp.transpose(wm_col)
    riota = jax.lax.broadcasted_iota(jnp.int32, (BC, 1), 0) + c * BC
    ohb = rk_row == riota
    oh = ohb.astype(jnp.bfloat16)
    xg = jnp.dot(oh, xbf_ref[...], preferred_element_type=jnp.float32)
    xg_ref[0] = xg.astype(jnp.bfloat16)
    wc_ref[0] = jnp.sum(jnp.where(ohb, wm_row, 0.0), axis=1, keepdims=True)


def _ffn_kernel(xg_ref, w1a_ref, w1b_ref, w2_ref, wc_ref, ct_ref):
    xg = xg_ref[0]
    h1a = jnp.dot(xg, w1a_ref[0], preferred_element_type=jnp.float32)
    h1b = jnp.dot(xg, w1b_ref[0], preferred_element_type=jnp.float32)
    act = (h1a * jax.nn.sigmoid(h1a) * h1b).astype(jnp.bfloat16)
    oe = jnp.dot(act, w2_ref[0], preferred_element_type=jnp.float32)
    ct_ref[0] = (oe * wc_ref[0]).astype(jnp.bfloat16)


def _scatter_kernel(rk_ref, ct_ref, out_ref):
    e = pl.program_id(0)
    rk = rk_ref[...]
    col = jax.lax.broadcasted_iota(jnp.int32, rk.shape, 1)
    rk_col = jnp.sum(jnp.where(col == e, rk, 0), axis=1, keepdims=True)
    riota = jax.lax.broadcasted_iota(jnp.int32, (1, CAP), 1)
    oht = (rk_col == riota).astype(jnp.bfloat16)
    delta = jnp.dot(oht, ct_ref[0], preferred_element_type=jnp.float32)

    @pl.when(e == 0)
    def _():
        out_ref[...] = delta

    @pl.when(e > 0)
    def _():
        out_ref[...] += delta


def kernel(x, Wg1, bg1, Wg2, W1, W2):
    w = pl.pallas_call(
        _gate_kernel,
        grid=(TB,),
        in_specs=[
            pl.BlockSpec((BT, H), lambda i: (i, 0)),
            pl.BlockSpec((H, GH), lambda i: (0, 0)),
            pl.BlockSpec((6, GH), lambda i: (0, 0)),
            pl.BlockSpec((1, GH), lambda i: (0, 0)),
            pl.BlockSpec((GH, E), lambda i: (0, 0)),
        ],
        out_specs=pl.BlockSpec((BT, E), lambda i: (i, 0)),
        out_shape=jax.ShapeDtypeStruct((T, E), jnp.float32),
    )(x, Wg1[:H], Wg1[H:], bg1[None, :], Wg2)

    wm, rk = pl.pallas_call(
        _route_kernel,
        out_shape=[
            jax.ShapeDtypeStruct((T, E), jnp.float32),
            jax.ShapeDtypeStruct((T, E), jnp.int32),
        ],
    )(w)

    xbf = x.astype(jnp.bfloat16)

    xg, wc = pl.pallas_call(
        _gather_kernel,
        grid=(E, CB),
        in_specs=[
            pl.BlockSpec((T, H), lambda e, c: (0, 0)),
            pl.BlockSpec((T, E), lambda e, c: (0, 0)),
            pl.BlockSpec((T, E), lambda e, c: (0, 0)),
        ],
        out_specs=[
            pl.BlockSpec((1, BC, H), lambda e, c: (e, c, 0)),
            pl.BlockSpec((1, BC, 1), lambda e, c: (e, c, 0)),
        ],
        out_shape=[
            jax.ShapeDtypeStruct((E, CAP, H), jnp.bfloat16),
            jax.ShapeDtypeStruct((E, CAP, 1), jnp.float32),
        ],
    )(xbf, rk, wm)

    w1a = W1[:, :, :INTER].astype(jnp.bfloat16)
    w1b = W1[:, :, INTER:].astype(jnp.bfloat16)
    w2b = W2.astype(jnp.bfloat16)

    ct = pl.pallas_call(
        _ffn_kernel,
        grid=(E, CB),
        in_specs=[
            pl.BlockSpec((1, BC, H), lambda e, c: (e, c, 0)),
            pl.BlockSpec((1, H, INTER), lambda e, c: (e, 0, 0)),
            pl.BlockSpec((1, H, INTER), lambda e, c: (e, 0, 0)),
            pl.BlockSpec((1, INTER, H), lambda e, c: (e, 0, 0)),
            pl.BlockSpec((1, BC, 1), lambda e, c: (e, c, 0)),
        ],
        out_specs=pl.BlockSpec((1, BC, H), lambda e, c: (e, c, 0)),
        out_shape=jax.ShapeDtypeStruct((E, CAP, H), jnp.bfloat16),
    )(xg, w1a, w1b, w2b, wc)

    out = pl.pallas_call(
        _scatter_kernel,
        grid=(E,),
        in_specs=[
            pl.BlockSpec((T, E), lambda e: (0, 0)),
            pl.BlockSpec((1, CAP, H), lambda e: (e, 0, 0)),
        ],
        out_specs=pl.BlockSpec((T, H), lambda e: (0, 0)),
        out_shape=jax.ShapeDtypeStruct((T, H), jnp.float32),
    )(rk, ct)

    return out
